# baseline (speedup 1.0000x reference)
import jax
import jax.numpy as jnp
from jax import lax
from jax.experimental import pallas as pl
from jax.experimental.pallas import tpu as pltpu
from jax.experimental.pallas import tpu_sc as plsc

LANES = 16
ROW = 128
NUM_WORKERS = 32
CHUNK_S = 16


def kernel(x, pos_table):
    batch, seq_len, d = x.shape
    rows_per_s = d // ROW
    x2 = x.reshape(batch * seq_len * rows_per_s, ROW)
    pos2 = pos_table.reshape(pos_table.shape[0] * rows_per_s, ROW)
    s_per_worker = seq_len // NUM_WORKERS
    n_chunks = s_per_worker // CHUNK_S
    chunk_rows = CHUNK_S * rows_per_s
    batch_stride = seq_len * rows_per_s

    mesh = plsc.VectorSubcoreMesh(core_axis_name="c", subcore_axis_name="s")

    @pl.kernel(
        mesh=mesh,
        out_type=jax.ShapeDtypeStruct(x2.shape, x2.dtype),
        scratch_types=[
            pltpu.VMEM((chunk_rows, ROW), jnp.float32),
            pltpu.VMEM((chunk_rows, ROW), jnp.float32),
            pltpu.VMEM((chunk_rows, ROW), jnp.float32),
            pltpu.VMEM((chunk_rows, ROW), jnp.float32),
            pltpu.VMEM((chunk_rows, ROW), jnp.float32),
        ],
    )
    def sc_add(x_hbm, pos_hbm, out_hbm, pos_v, xb0, xb1, xb2, xb3):
        nc = 2
        wid = lax.axis_index("s") * nc + lax.axis_index("c")
        xbufs = (xb0, xb1, xb2, xb3)

        def chunk_body(c, carry):
            p0 = (wid * s_per_worker + c * CHUNK_S) * rows_per_s
            pltpu.sync_copy(pos_hbm.at[pl.ds(p0, chunk_rows)], pos_v)
            for b in range(batch):
                pltpu.sync_copy(
                    x_hbm.at[pl.ds(p0 + b * batch_stride, chunk_rows)],
                    xbufs[b],
                )

            def add_body(j, carry2):
                for u in range(ROW // LANES):
                    col = pl.ds(u * LANES, LANES)
                    p = pos_v[j, col]
                    for b in range(batch):
                        xbufs[b][j, col] = xbufs[b][j, col] + p
                return carry2

            lax.fori_loop(0, chunk_rows, add_body, 0)
            for b in range(batch):
                pltpu.sync_copy(
                    xbufs[b],
                    out_hbm.at[pl.ds(p0 + b * batch_stride, chunk_rows)],
                )
            return carry

        lax.fori_loop(0, n_chunks, chunk_body, 0)

    out2 = sc_add(x2, pos2)
    return out2.reshape(batch, seq_len, d)


# speedup vs baseline: 1.0714x; 1.0714x over previous
import jax
import jax.numpy as jnp
from jax import lax
from jax.experimental import pallas as pl
from jax.experimental.pallas import tpu as pltpu
from jax.experimental.pallas import tpu_sc as plsc

LANES = 16
ROW = 128
NUM_WORKERS = 32
CHUNK_S = 16


def kernel(x, pos_table):
    batch, seq_len, d = x.shape
    rows_per_s = d // ROW
    x2 = x.reshape(batch * seq_len * rows_per_s, ROW)
    pos2 = pos_table.reshape(pos_table.shape[0] * rows_per_s, ROW)
    s_per_worker = seq_len // NUM_WORKERS
    n_chunks = s_per_worker // CHUNK_S
    chunk_rows = CHUNK_S * rows_per_s
    batch_stride = seq_len * rows_per_s

    mesh = plsc.VectorSubcoreMesh(core_axis_name="c", subcore_axis_name="s")

    @pl.kernel(
        mesh=mesh,
        out_type=jax.ShapeDtypeStruct(x2.shape, x2.dtype),
        scratch_types=[
            pltpu.VMEM((chunk_rows, ROW), jnp.float32),
            pltpu.VMEM((chunk_rows, ROW), jnp.float32),
            pltpu.VMEM((chunk_rows, ROW), jnp.float32),
            pltpu.VMEM((chunk_rows, ROW), jnp.float32),
            pltpu.VMEM((chunk_rows, ROW), jnp.float32),
            pltpu.SemaphoreType.DMA,
            pltpu.SemaphoreType.DMA,
        ],
    )
    def sc_add(x_hbm, pos_hbm, out_hbm, pos_v, xb0, xb1, xb2, xb3,
               in_sem, out_sem):
        nc = 2
        wid = lax.axis_index("s") * nc + lax.axis_index("c")
        xbufs = (xb0, xb1, xb2, xb3)

        def chunk_body(c, carry):
            p0 = (wid * s_per_worker + c * CHUNK_S) * rows_per_s
            handles = [
                pltpu.async_copy(pos_hbm.at[pl.ds(p0, chunk_rows)], pos_v,
                                 in_sem)
            ]
            for b in range(batch):
                handles.append(pltpu.async_copy(
                    x_hbm.at[pl.ds(p0 + b * batch_stride, chunk_rows)],
                    xbufs[b], in_sem))
            for h in handles:
                h.wait()

            def add_body(j, carry2):
                for u in range(ROW // LANES):
                    col = pl.ds(u * LANES, LANES)
                    p = pos_v[j, col]
                    for b in range(batch):
                        xbufs[b][j, col] = xbufs[b][j, col] + p
                return carry2

            lax.fori_loop(0, chunk_rows, add_body, 0)
            whandles = []
            for b in range(batch):
                whandles.append(pltpu.async_copy(
                    xbufs[b],
                    out_hbm.at[pl.ds(p0 + b * batch_stride, chunk_rows)],
                    out_sem))
            for h in whandles:
                h.wait()
            return carry

        lax.fori_loop(0, n_chunks, chunk_body, 0)

    out2 = sc_add(x2, pos2)
    return out2.reshape(batch, seq_len, d)


# speedup vs baseline: 6.0799x; 5.6746x over previous
import jax
import jax.numpy as jnp
from jax.experimental import pallas as pl
from jax.experimental.pallas import tpu as pltpu

BLOCK_S = 2048


def _add_kernel(x_ref, pos_ref, out_ref):
    out_ref[0] = x_ref[0] + pos_ref[...]


def kernel(x, pos_table):
    batch, seq_len, d = x.shape
    grid = (seq_len // BLOCK_S, batch)
    return pl.pallas_call(
        _add_kernel,
        grid=grid,
        in_specs=[
            pl.BlockSpec((1, BLOCK_S, d), lambda s, b: (b, s, 0)),
            pl.BlockSpec((BLOCK_S, d), lambda s, b: (s, 0)),
        ],
        out_specs=pl.BlockSpec((1, BLOCK_S, d), lambda s, b: (b, s, 0)),
        out_shape=jax.ShapeDtypeStruct((batch, seq_len, d), x.dtype),
        compiler_params=pltpu.CompilerParams(
            dimension_semantics=("parallel", "parallel"),
        ),
    )(x, pos_table)
